# baseline (device time: 55339 ns/iter reference)
import jax
import jax.numpy as jnp
from jax import lax
from jax.experimental import pallas as pl
from jax.experimental.pallas import tpu as pltpu


def kernel(partial, gamma):
    _, m_tot, d = partial.shape
    m_half = m_tot // 2
    p2d = partial.reshape(m_tot, d)
    g2d = gamma.reshape(1, d)

    def body(p_ref, g_ref, o_ref, comm_ref, send_sem, recv_sem):
        my_x = lax.axis_index("x")
        my_y = lax.axis_index("y")
        my_z = lax.axis_index("z")
        other_x = 1 - my_x
        partner = (other_x, my_y, my_z)

        barrier = pltpu.get_barrier_semaphore()
        pl.semaphore_signal(
            barrier, inc=1, device_id=partner,
            device_id_type=pl.DeviceIdType.MESH,
        )
        pl.semaphore_wait(barrier, 1)

        rdma = pltpu.make_async_remote_copy(
            src_ref=p_ref.at[pl.ds(other_x * m_half, m_half), :],
            dst_ref=comm_ref,
            send_sem=send_sem,
            recv_sem=recv_sem,
            device_id=partner,
            device_id_type=pl.DeviceIdType.MESH,
        )
        rdma.start()
        rdma.wait()

        y = p_ref[pl.ds(my_x * m_half, m_half), :] + comm_ref[:, :]
        ms = jnp.mean(y * y, axis=-1, keepdims=True)
        o_ref[:, :] = y * lax.rsqrt(ms + 1e-6) * g_ref[0, :][None, :]

    out = pl.pallas_call(
        body,
        out_shape=jax.ShapeDtypeStruct((m_half, d), jnp.float32),
        in_specs=[
            pl.BlockSpec(memory_space=pltpu.VMEM),
            pl.BlockSpec(memory_space=pltpu.VMEM),
        ],
        out_specs=pl.BlockSpec(memory_space=pltpu.VMEM),
        scratch_shapes=[
            pltpu.VMEM((m_half, d), jnp.float32),
            pltpu.SemaphoreType.DMA,
            pltpu.SemaphoreType.DMA,
        ],
        compiler_params=pltpu.CompilerParams(collective_id=0),
    )(p2d, g2d)
    return out


# device time: 37787 ns/iter; 1.4645x vs baseline; 1.4645x over previous
import jax
import jax.numpy as jnp
from jax import lax
from jax.experimental import pallas as pl
from jax.experimental.pallas import tpu as pltpu

NC = 8


def kernel(partial, gamma):
    _, m_tot, d = partial.shape
    m_half = m_tot // 2
    m_q = m_half // 2
    ch = m_q // NC
    p2d = partial.reshape(m_tot, d)
    g2d = gamma.reshape(1, d)

    def body(p_ref, g_ref, o_ref, loc_buf, xrecv_buf, loc_sem,
             xsend_sems, xrecv_sems, ysend_sems, yrecv_sems):
        my_x = lax.axis_index("x")
        my_y = lax.axis_index("y")
        my_z = lax.axis_index("z")
        xp = (1 - my_x, my_y, my_z)
        yp = (my_x, 1 - my_y, my_z)

        row_mine = my_x * m_half + my_y * m_q
        row_send = (1 - my_x) * m_half + my_y * m_q
        out_off = my_y * m_q

        barrier = pltpu.get_barrier_semaphore()
        for nbr in (xp, yp):
            pl.semaphore_signal(
                barrier, inc=1, device_id=nbr,
                device_id_type=pl.DeviceIdType.MESH,
            )
        pl.semaphore_wait(barrier, 2)

        loc_dma = pltpu.make_async_copy(
            p_ref.at[pl.ds(row_mine, m_q), :], loc_buf, loc_sem
        )
        loc_dma.start()

        xrdma = []
        for k in range(NC):
            r = pltpu.make_async_remote_copy(
                src_ref=p_ref.at[pl.ds(row_send + k * ch, ch), :],
                dst_ref=xrecv_buf.at[k],
                send_sem=xsend_sems.at[k],
                recv_sem=xrecv_sems.at[k],
                device_id=xp,
                device_id_type=pl.DeviceIdType.MESH,
            )
            r.start()
            xrdma.append(r)

        loc_dma.wait()

        yrdma = []
        for k in range(NC):
            xrdma[k].wait_recv()
            y = loc_buf[pl.ds(k * ch, ch), :] + xrecv_buf[k, :, :]
            ms = jnp.mean(y * y, axis=-1, keepdims=True)
            o_ref[pl.ds(out_off + k * ch, ch), :] = (
                y * lax.rsqrt(ms + 1e-6) * g_ref[0, :][None, :]
            )
            r = pltpu.make_async_remote_copy(
                src_ref=o_ref.at[pl.ds(out_off + k * ch, ch), :],
                dst_ref=o_ref.at[pl.ds(out_off + k * ch, ch), :],
                send_sem=ysend_sems.at[k],
                recv_sem=yrecv_sems.at[k],
                device_id=yp,
                device_id_type=pl.DeviceIdType.MESH,
            )
            r.start()
            yrdma.append(r)

        for k in range(NC):
            yrdma[k].wait_recv()
            xrdma[k].wait_send()
            yrdma[k].wait_send()

    out = pl.pallas_call(
        body,
        out_shape=jax.ShapeDtypeStruct((m_half, d), jnp.float32),
        in_specs=[
            pl.BlockSpec(memory_space=pl.ANY),
            pl.BlockSpec(memory_space=pltpu.VMEM),
        ],
        out_specs=pl.BlockSpec(memory_space=pltpu.VMEM),
        scratch_shapes=[
            pltpu.VMEM((m_q, d), jnp.float32),
            pltpu.VMEM((NC, ch, d), jnp.float32),
            pltpu.SemaphoreType.DMA,
            pltpu.SemaphoreType.DMA((NC,)),
            pltpu.SemaphoreType.DMA((NC,)),
            pltpu.SemaphoreType.DMA((NC,)),
            pltpu.SemaphoreType.DMA((NC,)),
        ],
        compiler_params=pltpu.CompilerParams(collective_id=0),
    )(p2d, g2d)
    return out
